# baseline (speedup 1.0000x reference)
import functools

import jax
import jax.numpy as jnp
from jax import lax
from jax.experimental import pallas as pl
from jax.experimental.pallas import tpu as pltpu
from jax.experimental.pallas import tpu_sc as plsc

_B, _N = 32, 1024
_R = 256
_L = 16
_V = _N // _L
_D0 = 16


def _sc_sort_body(scores_hbm, sorted_hbm, xb, yb, ob):
    wid = lax.axis_index("s") * 2 + lax.axis_index("c")
    pltpu.sync_copy(scores_hbm.at[wid], xb.at[pl.ds(_D0, _N)])

    def _init(i, _):
        off = _D0 + i * _L
        v = -xb[pl.ds(off, _L)]
        sv, _ = plsc.sort_key_val(v, v)
        xb[pl.ds(off, _L)] = sv
        return 0

    lax.fori_loop(0, _V, _init, 0)

    m = _L
    while m < _N:
        k = m // _L

        def _pair(p, _, m=m, k=k):
            base = _D0 + p * 2 * m
            for i in range(k):
                a = xb[pl.ds(base + i * _L, _L)]
                b = xb[pl.ds(base + m + (k - 1 - i) * _L, _L)]
                rb = lax.rev(b, (0,))
                yb[pl.ds(base + i * _L, _L)] = jnp.minimum(a, rb)
                yb[pl.ds(base + m + i * _L, _L)] = jnp.maximum(a, rb)
            dd = k // 2
            while dd >= 1:
                for g in range(k // dd):
                    for i in range(dd):
                        u = base + (g * 2 * dd + i) * _L
                        w = u + dd * _L
                        av = yb[pl.ds(u, _L)]
                        bv = yb[pl.ds(w, _L)]
                        yb[pl.ds(u, _L)] = jnp.minimum(av, bv)
                        yb[pl.ds(w, _L)] = jnp.maximum(av, bv)
                dd //= 2
            for i in range(2 * k):
                v = yb[pl.ds(base + i * _L, _L)]
                sv, _ = plsc.sort_key_val(v, v)
                xb[pl.ds(base + i * _L, _L)] = sv
            return 0

        lax.fori_loop(0, _N // (2 * m), _pair, 0)
        m *= 2

    def _fin(i, _):
        ob[pl.ds(i * _L, _L)] = -xb[pl.ds(_D0 + i * _L, _L)]
        return 0

    lax.fori_loop(0, _V, _fin, 0)
    pltpu.sync_copy(ob, sorted_hbm.at[wid])


_sc_sort = functools.partial(
    pl.kernel,
    out_type=jax.ShapeDtypeStruct((_B, _N), jnp.float32),
    mesh=plsc.VectorSubcoreMesh(core_axis_name="c", subcore_axis_name="s"),
    compiler_params=pltpu.CompilerParams(
        needs_layout_passes=False, use_tc_tiling_on_sc=False),
    scratch_types=[
        pltpu.VMEM((_D0 + _N,), jnp.float32),
        pltpu.VMEM((_D0 + _N,), jnp.float32),
        pltpu.VMEM((_N,), jnp.float32),
    ],
)(_sc_sort_body)


def _build_block(scores_ref, s_ref, out_ref):
    x = scores_ref[0]
    s = s_ref[0]
    out_ref[...] = jnp.where(x == s, 1.0, 0.0)[None]


def kernel(scores):
    sorted_desc = _sc_sort(scores)
    return pl.pallas_call(
        _build_block,
        grid=(_B, _N // _R),
        in_specs=[
            pl.BlockSpec((1, 1, _N), lambda b, r: (b, 0, 0)),
            pl.BlockSpec((1, _R, 1), lambda b, r: (b, r, 0)),
        ],
        out_specs=pl.BlockSpec((1, _R, _N), lambda b, r: (b, r, 0)),
        out_shape=jax.ShapeDtypeStruct((_B, _N, _N), scores.dtype),
    )(scores[:, None, :], sorted_desc[:, :, None])


# speedup vs baseline: 1.2874x; 1.2874x over previous
import functools

import jax
import jax.numpy as jnp
from jax import lax
from jax.experimental import pallas as pl
from jax.experimental.pallas import tpu as pltpu
from jax.experimental.pallas import tpu_sc as plsc

_B, _N = 32, 1024
_R = 512
_L = 16
_V = _N // _L
_D0 = 16


def _sc_sort_body(scores_hbm, sorted_hbm, xb, yb, ob):
    wid = lax.axis_index("s") * 2 + lax.axis_index("c")
    pltpu.sync_copy(scores_hbm.at[wid], xb.at[pl.ds(_D0, _N)])

    def _init(i, _):
        off = _D0 + i * _L
        v = -xb[pl.ds(off, _L)]
        sv, _ = plsc.sort_key_val(v, v)
        xb[pl.ds(off, _L)] = sv
        return 0

    lax.fori_loop(0, _V, _init, 0)

    m = _L
    while m < _N:
        k = m // _L

        def _pair(p, _, m=m, k=k):
            base = _D0 + p * 2 * m
            for i in range(k):
                a = xb[pl.ds(base + i * _L, _L)]
                b = xb[pl.ds(base + m + (k - 1 - i) * _L, _L)]
                rb = lax.rev(b, (0,))
                yb[pl.ds(base + i * _L, _L)] = jnp.minimum(a, rb)
                yb[pl.ds(base + m + i * _L, _L)] = jnp.maximum(a, rb)
            dd = k // 2
            while dd >= 1:
                for g in range(k // dd):
                    for i in range(dd):
                        u = base + (g * 2 * dd + i) * _L
                        w = u + dd * _L
                        av = yb[pl.ds(u, _L)]
                        bv = yb[pl.ds(w, _L)]
                        yb[pl.ds(u, _L)] = jnp.minimum(av, bv)
                        yb[pl.ds(w, _L)] = jnp.maximum(av, bv)
                dd //= 2
            for i in range(2 * k):
                v = yb[pl.ds(base + i * _L, _L)]
                sv, _ = plsc.sort_key_val(v, v)
                xb[pl.ds(base + i * _L, _L)] = sv
            return 0

        lax.fori_loop(0, _N // (2 * m), _pair, 0)
        m *= 2

    def _fin(i, _):
        ob[pl.ds(i * _L, _L)] = -xb[pl.ds(_D0 + i * _L, _L)]
        return 0

    lax.fori_loop(0, _V, _fin, 0)
    pltpu.sync_copy(ob, sorted_hbm.at[wid])


_sc_sort = functools.partial(
    pl.kernel,
    out_type=jax.ShapeDtypeStruct((_B, _N), jnp.float32),
    mesh=plsc.VectorSubcoreMesh(core_axis_name="c", subcore_axis_name="s"),
    compiler_params=pltpu.CompilerParams(
        needs_layout_passes=False, use_tc_tiling_on_sc=False),
    scratch_types=[
        pltpu.VMEM((_D0 + _N,), jnp.float32),
        pltpu.VMEM((_D0 + _N,), jnp.float32),
        pltpu.VMEM((_N,), jnp.float32),
    ],
)(_sc_sort_body)


def _build_block(scores_ref, s_ref, out_ref):
    x = scores_ref[0]
    s = s_ref[0]
    out_ref[...] = jnp.where(x == s, 1.0, 0.0)[None]


def kernel(scores):
    sorted_desc = _sc_sort(scores)
    return pl.pallas_call(
        _build_block,
        grid=(_B, _N // _R),
        in_specs=[
            pl.BlockSpec((1, 1, _N), lambda b, r: (b, 0, 0)),
            pl.BlockSpec((1, _R, 1), lambda b, r: (b, r, 0)),
        ],
        out_specs=pl.BlockSpec((1, _R, _N), lambda b, r: (b, r, 0)),
        out_shape=jax.ShapeDtypeStruct((_B, _N, _N), scores.dtype),
    )(scores[:, None, :], sorted_desc[:, :, None])


# speedup vs baseline: 1.5110x; 1.1737x over previous
import functools

import jax
import jax.numpy as jnp
from jax import lax
from jax.experimental import pallas as pl
from jax.experimental.pallas import tpu as pltpu
from jax.experimental.pallas import tpu_sc as plsc

_B, _N = 32, 1024
_R = 1024
_L = 16
_V = _N // _L
_D0 = 16


def _sc_sort_body(scores_hbm, sorted_hbm, xb, yb, ob):
    wid = lax.axis_index("s") * 2 + lax.axis_index("c")
    pltpu.sync_copy(scores_hbm.at[wid], xb.at[pl.ds(_D0, _N)])

    def _init(i, _):
        off = _D0 + i * _L
        v = -xb[pl.ds(off, _L)]
        sv, _ = plsc.sort_key_val(v, v)
        xb[pl.ds(off, _L)] = sv
        return 0

    lax.fori_loop(0, _V, _init, 0)

    m = _L
    while m < _N:
        k = m // _L

        def _pair(p, _, m=m, k=k):
            base = _D0 + p * 2 * m
            for i in range(k):
                a = xb[pl.ds(base + i * _L, _L)]
                b = xb[pl.ds(base + m + (k - 1 - i) * _L, _L)]
                rb = lax.rev(b, (0,))
                yb[pl.ds(base + i * _L, _L)] = jnp.minimum(a, rb)
                yb[pl.ds(base + m + i * _L, _L)] = jnp.maximum(a, rb)
            dd = k // 2
            while dd >= 1:
                for g in range(k // dd):
                    for i in range(dd):
                        u = base + (g * 2 * dd + i) * _L
                        w = u + dd * _L
                        av = yb[pl.ds(u, _L)]
                        bv = yb[pl.ds(w, _L)]
                        yb[pl.ds(u, _L)] = jnp.minimum(av, bv)
                        yb[pl.ds(w, _L)] = jnp.maximum(av, bv)
                dd //= 2
            for i in range(2 * k):
                v = yb[pl.ds(base + i * _L, _L)]
                sv, _ = plsc.sort_key_val(v, v)
                xb[pl.ds(base + i * _L, _L)] = sv
            return 0

        lax.fori_loop(0, _N // (2 * m), _pair, 0)
        m *= 2

    def _fin(i, _):
        ob[pl.ds(i * _L, _L)] = -xb[pl.ds(_D0 + i * _L, _L)]
        return 0

    lax.fori_loop(0, _V, _fin, 0)
    pltpu.sync_copy(ob, sorted_hbm.at[wid])


_sc_sort = functools.partial(
    pl.kernel,
    out_type=jax.ShapeDtypeStruct((_B, _N), jnp.float32),
    mesh=plsc.VectorSubcoreMesh(core_axis_name="c", subcore_axis_name="s"),
    compiler_params=pltpu.CompilerParams(
        needs_layout_passes=False, use_tc_tiling_on_sc=False),
    scratch_types=[
        pltpu.VMEM((_D0 + _N,), jnp.float32),
        pltpu.VMEM((_D0 + _N,), jnp.float32),
        pltpu.VMEM((_N,), jnp.float32),
    ],
)(_sc_sort_body)


def _build_block(scores_ref, s_ref, out_ref):
    x = scores_ref[0]
    s = s_ref[0]
    out_ref[...] = jnp.where(x == s, 1.0, 0.0)[None]


def kernel(scores):
    sorted_desc = _sc_sort(scores)
    return pl.pallas_call(
        _build_block,
        grid=(_B, _N // _R),
        in_specs=[
            pl.BlockSpec((1, 1, _N), lambda b, r: (b, 0, 0)),
            pl.BlockSpec((1, _R, 1), lambda b, r: (b, r, 0)),
        ],
        out_specs=pl.BlockSpec((1, _R, _N), lambda b, r: (b, r, 0)),
        out_shape=jax.ShapeDtypeStruct((_B, _N, _N), scores.dtype),
    )(scores[:, None, :], sorted_desc[:, :, None])


# speedup vs baseline: 1.5792x; 1.0451x over previous
import functools

import jax
import jax.numpy as jnp
from jax import lax
from jax.experimental import pallas as pl
from jax.experimental.pallas import tpu as pltpu
from jax.experimental.pallas import tpu_sc as plsc

_B, _N = 32, 1024
_R = 1024
_L = 16
_V = _N // _L
_D0 = 16


def _sc_sort_body(scores_hbm, sorted_hbm, xb, yb, ob):
    wid = lax.axis_index("s") * 2 + lax.axis_index("c")
    pltpu.sync_copy(scores_hbm.at[wid], xb.at[pl.ds(_D0, _N)])

    def _init(i, _):
        off = _D0 + i * _L
        v = -xb[pl.ds(off, _L)]
        sv, _ = plsc.sort_key_val(v, v)
        xb[pl.ds(off, _L)] = sv
        return 0

    lax.fori_loop(0, _V, _init, 0)

    m = _L
    while m < _N:
        k = m // _L

        def _pair(p, _, m=m, k=k):
            base = _D0 + p * 2 * m
            for i in range(k):
                a = xb[pl.ds(base + i * _L, _L)]
                b = xb[pl.ds(base + m + (k - 1 - i) * _L, _L)]
                rb = lax.rev(b, (0,))
                yb[pl.ds(base + i * _L, _L)] = jnp.minimum(a, rb)
                yb[pl.ds(base + m + i * _L, _L)] = jnp.maximum(a, rb)
            dd = k // 2
            while dd >= 1:
                for g in range(k // dd):
                    for i in range(dd):
                        u = base + (g * 2 * dd + i) * _L
                        w = u + dd * _L
                        av = yb[pl.ds(u, _L)]
                        bv = yb[pl.ds(w, _L)]
                        yb[pl.ds(u, _L)] = jnp.minimum(av, bv)
                        yb[pl.ds(w, _L)] = jnp.maximum(av, bv)
                dd //= 2
            for i in range(2 * k):
                v = yb[pl.ds(base + i * _L, _L)]
                sv, _ = plsc.sort_key_val(v, v)
                xb[pl.ds(base + i * _L, _L)] = sv
            return 0

        lax.fori_loop(0, _N // (2 * m), _pair, 0)
        m *= 2

    def _fin(i, _):
        ob[pl.ds(i * _L, _L)] = -xb[pl.ds(_D0 + i * _L, _L)]
        return 0

    lax.fori_loop(0, _V, _fin, 0)
    pltpu.sync_copy(ob, sorted_hbm.at[wid])


_sc_sort = functools.partial(
    pl.kernel,
    out_type=jax.ShapeDtypeStruct((_B, _N), jnp.float32),
    mesh=plsc.VectorSubcoreMesh(core_axis_name="c", subcore_axis_name="s"),
    compiler_params=pltpu.CompilerParams(
        needs_layout_passes=False, use_tc_tiling_on_sc=False),
    scratch_types=[
        pltpu.VMEM((_D0 + _N,), jnp.float32),
        pltpu.VMEM((_D0 + _N,), jnp.float32),
        pltpu.VMEM((_N,), jnp.float32),
    ],
)(_sc_sort_body)


_BB = 2


def _build_block(scores_ref, s_ref, out_ref):
    x = scores_ref[...]
    s = s_ref[...]
    out_ref[...] = jnp.where(x == s, 1.0, 0.0)


def kernel(scores):
    sorted_desc = _sc_sort(scores)
    return pl.pallas_call(
        _build_block,
        grid=(_B // _BB,),
        in_specs=[
            pl.BlockSpec((_BB, 1, _N), lambda b: (b, 0, 0)),
            pl.BlockSpec((_BB, _N, 1), lambda b: (b, 0, 0)),
        ],
        out_specs=pl.BlockSpec((_BB, _N, _N), lambda b: (b, 0, 0)),
        out_shape=jax.ShapeDtypeStruct((_B, _N, _N), scores.dtype),
    )(scores[:, None, :], sorted_desc[:, :, None])


# speedup vs baseline: 1.5913x; 1.0077x over previous
import functools

import jax
import jax.numpy as jnp
from jax import lax
from jax.experimental import pallas as pl
from jax.experimental.pallas import tpu as pltpu
from jax.experimental.pallas import tpu_sc as plsc

_B, _N = 32, 1024
_R = 1024
_L = 16
_V = _N // _L
_D0 = 16


def _sc_sort_body(scores_hbm, sorted_hbm, xb, yb, ob):
    wid = lax.axis_index("s") * 2 + lax.axis_index("c")
    pltpu.sync_copy(scores_hbm.at[wid], xb.at[pl.ds(_D0, _N)])

    def _init(i, _):
        off = _D0 + i * _L
        v = -xb[pl.ds(off, _L)]
        sv, _ = plsc.sort_key_val(v, v)
        xb[pl.ds(off, _L)] = sv
        return 0

    lax.fori_loop(0, _V, _init, 0)

    m = _L
    while m < _N:
        k = m // _L

        def _pair(p, _, m=m, k=k):
            base = _D0 + p * 2 * m
            for i in range(k):
                a = xb[pl.ds(base + i * _L, _L)]
                b = xb[pl.ds(base + m + (k - 1 - i) * _L, _L)]
                rb = lax.rev(b, (0,))
                yb[pl.ds(base + i * _L, _L)] = jnp.minimum(a, rb)
                yb[pl.ds(base + m + i * _L, _L)] = jnp.maximum(a, rb)
            dd = k // 2
            while dd >= 1:
                for g in range(k // dd):
                    for i in range(dd):
                        u = base + (g * 2 * dd + i) * _L
                        w = u + dd * _L
                        av = yb[pl.ds(u, _L)]
                        bv = yb[pl.ds(w, _L)]
                        yb[pl.ds(u, _L)] = jnp.minimum(av, bv)
                        yb[pl.ds(w, _L)] = jnp.maximum(av, bv)
                dd //= 2
            for i in range(2 * k):
                v = yb[pl.ds(base + i * _L, _L)]
                sv, _ = plsc.sort_key_val(v, v)
                xb[pl.ds(base + i * _L, _L)] = sv
            return 0

        lax.fori_loop(0, _N // (2 * m), _pair, 0)
        m *= 2

    def _fin(i, _):
        ob[pl.ds(i * _L, _L)] = -xb[pl.ds(_D0 + i * _L, _L)]
        return 0

    lax.fori_loop(0, _V, _fin, 0)
    pltpu.sync_copy(ob, sorted_hbm.at[wid])


_sc_sort = functools.partial(
    pl.kernel,
    out_type=jax.ShapeDtypeStruct((_B, _N), jnp.float32),
    mesh=plsc.VectorSubcoreMesh(core_axis_name="c", subcore_axis_name="s"),
    compiler_params=pltpu.CompilerParams(
        needs_layout_passes=False, use_tc_tiling_on_sc=False),
    scratch_types=[
        pltpu.VMEM((_D0 + _N,), jnp.float32),
        pltpu.VMEM((_D0 + _N,), jnp.float32),
        pltpu.VMEM((_N,), jnp.float32),
    ],
)(_sc_sort_body)


_BB = 4


def _build_block(scores_ref, s_ref, out_ref):
    x = scores_ref[...]
    s = s_ref[...]
    out_ref[...] = jnp.where(x == s, 1.0, 0.0)


def kernel(scores):
    sorted_desc = _sc_sort(scores)
    return pl.pallas_call(
        _build_block,
        grid=(_B // _BB,),
        in_specs=[
            pl.BlockSpec((_BB, 1, _N), lambda b: (b, 0, 0)),
            pl.BlockSpec((_BB, _N, 1), lambda b: (b, 0, 0)),
        ],
        out_specs=pl.BlockSpec((_BB, _N, _N), lambda b: (b, 0, 0)),
        out_shape=jax.ShapeDtypeStruct((_B, _N, _N), scores.dtype),
    )(scores[:, None, :], sorted_desc[:, :, None])


# speedup vs baseline: 1.9348x; 1.2159x over previous
import functools

import jax
import jax.numpy as jnp
from jax import lax
from jax.experimental import pallas as pl
from jax.experimental.pallas import tpu as pltpu
from jax.experimental.pallas import tpu_sc as plsc

_B, _N = 32, 1024
_L = 16
_V = _N // _L
_D0 = 16


def _sc_rank_body(scores_hbm, rank_hbm, xb, yb, vxb, vyb, rb_):
    wid = lax.axis_index("s") * 2 + lax.axis_index("c")
    pltpu.sync_copy(scores_hbm.at[wid], xb.at[pl.ds(_D0, _N)])

    def _init(i, _):
        off = _D0 + i * _L
        k = -xb[pl.ds(off, _L)]
        v = lax.iota(jnp.int32, _L) + i * _L
        sk, sv = plsc.sort_key_val(k, v)
        xb[pl.ds(off, _L)] = sk
        vxb[pl.ds(off, _L)] = sv
        return 0

    lax.fori_loop(0, _V, _init, 0)

    m = _L
    while m < _N:
        k = m // _L

        def _pair(p, _, m=m, k=k):
            base = _D0 + p * 2 * m
            for i in range(k):
                ak = xb[pl.ds(base + i * _L, _L)]
                bk = lax.rev(xb[pl.ds(base + m + (k - 1 - i) * _L, _L)], (0,))
                av = vxb[pl.ds(base + i * _L, _L)]
                bv = lax.rev(vxb[pl.ds(base + m + (k - 1 - i) * _L, _L)], (0,))
                sel = ak <= bk
                yb[pl.ds(base + i * _L, _L)] = jnp.minimum(ak, bk)
                yb[pl.ds(base + m + i * _L, _L)] = jnp.maximum(ak, bk)
                vyb[pl.ds(base + i * _L, _L)] = jnp.where(sel, av, bv)
                vyb[pl.ds(base + m + i * _L, _L)] = jnp.where(sel, bv, av)
            dd = k // 2
            while dd >= 1:
                for g in range(k // dd):
                    for i in range(dd):
                        u = base + (g * 2 * dd + i) * _L
                        w = u + dd * _L
                        ak = yb[pl.ds(u, _L)]
                        bk = yb[pl.ds(w, _L)]
                        av = vyb[pl.ds(u, _L)]
                        bv = vyb[pl.ds(w, _L)]
                        sel = ak <= bk
                        yb[pl.ds(u, _L)] = jnp.minimum(ak, bk)
                        yb[pl.ds(w, _L)] = jnp.maximum(ak, bk)
                        vyb[pl.ds(u, _L)] = jnp.where(sel, av, bv)
                        vyb[pl.ds(w, _L)] = jnp.where(sel, bv, av)
                dd //= 2
            for i in range(2 * k):
                kk = yb[pl.ds(base + i * _L, _L)]
                vv = vyb[pl.ds(base + i * _L, _L)]
                sk, sv = plsc.sort_key_val(kk, vv)
                xb[pl.ds(base + i * _L, _L)] = sk
                vxb[pl.ds(base + i * _L, _L)] = sv
            return 0

        lax.fori_loop(0, _N // (2 * m), _pair, 0)
        m *= 2

    def _fin(i, _):
        pv = vxb[pl.ds(_D0 + i * _L, _L)]
        rv = lax.iota(jnp.int32, _L) + i * _L
        plsc.store_scatter(rb_, [pv], rv)
        return 0

    lax.fori_loop(0, _V, _fin, 0)
    pltpu.sync_copy(rb_, rank_hbm.at[wid])


_sc_rank = functools.partial(
    pl.kernel,
    out_type=jax.ShapeDtypeStruct((_B, _N), jnp.int32),
    mesh=plsc.VectorSubcoreMesh(core_axis_name="c", subcore_axis_name="s"),
    compiler_params=pltpu.CompilerParams(
        needs_layout_passes=False, use_tc_tiling_on_sc=False),
    scratch_types=[
        pltpu.VMEM((_D0 + _N,), jnp.float32),
        pltpu.VMEM((_D0 + _N,), jnp.float32),
        pltpu.VMEM((_D0 + _N,), jnp.int32),
        pltpu.VMEM((_D0 + _N,), jnp.int32),
        pltpu.VMEM((_N,), jnp.int32),
    ],
)(_sc_rank_body)


_BB = 4


def _build_block(rank_ref, out_ref):
    rank = rank_ref[...]
    r = lax.broadcasted_iota(jnp.int32, (_BB, _N, _N), 1)
    out_ref[...] = jnp.where(rank == r, 1.0, 0.0)


def kernel(scores):
    rank = _sc_rank(scores)
    return pl.pallas_call(
        _build_block,
        grid=(_B // _BB,),
        in_specs=[pl.BlockSpec((_BB, 1, _N), lambda b: (b, 0, 0))],
        out_specs=pl.BlockSpec((_BB, _N, _N), lambda b: (b, 0, 0)),
        out_shape=jax.ShapeDtypeStruct((_B, _N, _N), scores.dtype),
    )(rank[:, None, :])


# speedup vs baseline: 1.9373x; 1.0013x over previous
import functools

import jax
import jax.numpy as jnp
from jax import lax
from jax.experimental import pallas as pl
from jax.experimental.pallas import tpu as pltpu
from jax.experimental.pallas import tpu_sc as plsc

_B, _N = 32, 1024
_L = 16
_V = _N // _L
_D0 = 16


def _sc_rank_body(scores_hbm, rank_hbm, xb, yb, vxb, vyb, rb_):
    wid = lax.axis_index("s") * 2 + lax.axis_index("c")
    pltpu.sync_copy(scores_hbm.at[wid], xb.at[pl.ds(_D0, _N)])

    @plsc.parallel_loop(0, _V)
    def _init(i):
        off = _D0 + i * _L
        k = -xb[pl.ds(off, _L)]
        v = lax.iota(jnp.int32, _L) + i * _L
        sk, sv = plsc.sort_key_val(k, v)
        xb[pl.ds(off, _L)] = sk
        vxb[pl.ds(off, _L)] = sv

    m = _L
    while m < _N:
        k = m // _L

        @plsc.parallel_loop(0, _N // (2 * m))
        def _pair(p, m=m, k=k):
            base = _D0 + p * 2 * m
            for i in range(k):
                ak = xb[pl.ds(base + i * _L, _L)]
                bk = lax.rev(xb[pl.ds(base + m + (k - 1 - i) * _L, _L)], (0,))
                av = vxb[pl.ds(base + i * _L, _L)]
                bv = lax.rev(vxb[pl.ds(base + m + (k - 1 - i) * _L, _L)], (0,))
                sel = ak <= bk
                yb[pl.ds(base + i * _L, _L)] = jnp.minimum(ak, bk)
                yb[pl.ds(base + m + i * _L, _L)] = jnp.maximum(ak, bk)
                vyb[pl.ds(base + i * _L, _L)] = jnp.where(sel, av, bv)
                vyb[pl.ds(base + m + i * _L, _L)] = jnp.where(sel, bv, av)
            dd = k // 2
            while dd >= 1:
                for g in range(k // dd):
                    for i in range(dd):
                        u = base + (g * 2 * dd + i) * _L
                        w = u + dd * _L
                        ak = yb[pl.ds(u, _L)]
                        bk = yb[pl.ds(w, _L)]
                        av = vyb[pl.ds(u, _L)]
                        bv = vyb[pl.ds(w, _L)]
                        sel = ak <= bk
                        yb[pl.ds(u, _L)] = jnp.minimum(ak, bk)
                        yb[pl.ds(w, _L)] = jnp.maximum(ak, bk)
                        vyb[pl.ds(u, _L)] = jnp.where(sel, av, bv)
                        vyb[pl.ds(w, _L)] = jnp.where(sel, bv, av)
                dd //= 2
            for i in range(2 * k):
                kk = yb[pl.ds(base + i * _L, _L)]
                vv = vyb[pl.ds(base + i * _L, _L)]
                sk, sv = plsc.sort_key_val(kk, vv)
                xb[pl.ds(base + i * _L, _L)] = sk
                vxb[pl.ds(base + i * _L, _L)] = sv
        m *= 2

    @plsc.parallel_loop(0, _V)
    def _fin(i):
        pv = vxb[pl.ds(_D0 + i * _L, _L)]
        rv = lax.iota(jnp.int32, _L) + i * _L
        plsc.store_scatter(rb_, [pv], rv)
    pltpu.sync_copy(rb_, rank_hbm.at[wid])


_sc_rank = functools.partial(
    pl.kernel,
    out_type=jax.ShapeDtypeStruct((_B, _N), jnp.int32),
    mesh=plsc.VectorSubcoreMesh(core_axis_name="c", subcore_axis_name="s"),
    compiler_params=pltpu.CompilerParams(
        needs_layout_passes=False, use_tc_tiling_on_sc=False),
    scratch_types=[
        pltpu.VMEM((_D0 + _N,), jnp.float32),
        pltpu.VMEM((_D0 + _N,), jnp.float32),
        pltpu.VMEM((_D0 + _N,), jnp.int32),
        pltpu.VMEM((_D0 + _N,), jnp.int32),
        pltpu.VMEM((_N,), jnp.int32),
    ],
)(_sc_rank_body)


_BB = 4


def _build_block(rank_ref, out_ref):
    rank = rank_ref[...]
    r = lax.broadcasted_iota(jnp.int32, (_BB, _N, _N), 1)
    out_ref[...] = jnp.where(rank == r, 1.0, 0.0)


def kernel(scores):
    rank = _sc_rank(scores)
    return pl.pallas_call(
        _build_block,
        grid=(_B // _BB,),
        in_specs=[pl.BlockSpec((_BB, 1, _N), lambda b: (b, 0, 0))],
        out_specs=pl.BlockSpec((_BB, _N, _N), lambda b: (b, 0, 0)),
        out_shape=jax.ShapeDtypeStruct((_B, _N, _N), scores.dtype),
    )(rank[:, None, :])


# speedup vs baseline: 1.9696x; 1.0167x over previous
import functools

import jax
import jax.numpy as jnp
from jax import lax
from jax.experimental import pallas as pl
from jax.experimental.pallas import tpu as pltpu
from jax.experimental.pallas import tpu_sc as plsc

_B, _N = 32, 1024
_L = 16
_V = _N // _L
_D0 = 16


def _sc_rank_body(scores_hbm, rank_hbm, xb, yb, vxb, vyb, rb_):
    wid = lax.axis_index("s") * 2 + lax.axis_index("c")
    pltpu.sync_copy(scores_hbm.at[wid], xb.at[pl.ds(_D0, _N)])

    @plsc.parallel_loop(0, _V)
    def _init(i):
        off = _D0 + i * _L
        k = -xb[pl.ds(off, _L)]
        v = lax.iota(jnp.int32, _L) + i * _L
        sk, sv = plsc.sort_key_val(k, v)
        xb[pl.ds(off, _L)] = sk
        vxb[pl.ds(off, _L)] = sv

    m = _L
    while m < _N:
        k = m // _L

        @plsc.parallel_loop(0, _N // (2 * m))
        def _pair(p, m=m, k=k):
            base = _D0 + p * 2 * m
            for i in range(k):
                ak = xb[pl.ds(base + i * _L, _L)]
                bk = lax.rev(xb[pl.ds(base + m + (k - 1 - i) * _L, _L)], (0,))
                av = vxb[pl.ds(base + i * _L, _L)]
                bv = lax.rev(vxb[pl.ds(base + m + (k - 1 - i) * _L, _L)], (0,))
                sel = ak <= bk
                yb[pl.ds(base + i * _L, _L)] = jnp.minimum(ak, bk)
                yb[pl.ds(base + m + i * _L, _L)] = jnp.maximum(ak, bk)
                vyb[pl.ds(base + i * _L, _L)] = jnp.where(sel, av, bv)
                vyb[pl.ds(base + m + i * _L, _L)] = jnp.where(sel, bv, av)
            dd = k // 2
            while dd >= 1:
                for g in range(k // dd):
                    for i in range(dd):
                        u = base + (g * 2 * dd + i) * _L
                        w = u + dd * _L
                        ak = yb[pl.ds(u, _L)]
                        bk = yb[pl.ds(w, _L)]
                        av = vyb[pl.ds(u, _L)]
                        bv = vyb[pl.ds(w, _L)]
                        sel = ak <= bk
                        yb[pl.ds(u, _L)] = jnp.minimum(ak, bk)
                        yb[pl.ds(w, _L)] = jnp.maximum(ak, bk)
                        vyb[pl.ds(u, _L)] = jnp.where(sel, av, bv)
                        vyb[pl.ds(w, _L)] = jnp.where(sel, bv, av)
                dd //= 2
            for i in range(2 * k):
                kk = yb[pl.ds(base + i * _L, _L)]
                vv = vyb[pl.ds(base + i * _L, _L)]
                sk, sv = plsc.sort_key_val(kk, vv)
                xb[pl.ds(base + i * _L, _L)] = sk
                vxb[pl.ds(base + i * _L, _L)] = sv
        m *= 2

    @plsc.parallel_loop(0, _V)
    def _fin(i):
        pv = vxb[pl.ds(_D0 + i * _L, _L)]
        rv = lax.iota(jnp.int32, _L) + i * _L
        plsc.store_scatter(rb_, [pv], rv)
    pltpu.sync_copy(rb_, rank_hbm.at[wid])


_sc_rank = functools.partial(
    pl.kernel,
    out_type=jax.ShapeDtypeStruct((_B, _N), jnp.int32),
    mesh=plsc.VectorSubcoreMesh(core_axis_name="c", subcore_axis_name="s"),
    compiler_params=pltpu.CompilerParams(
        needs_layout_passes=False, use_tc_tiling_on_sc=False),
    scratch_types=[
        pltpu.VMEM((_D0 + _N,), jnp.float32),
        pltpu.VMEM((_D0 + _N,), jnp.float32),
        pltpu.VMEM((_D0 + _N,), jnp.int32),
        pltpu.VMEM((_D0 + _N,), jnp.int32),
        pltpu.VMEM((_N,), jnp.int32),
    ],
)(_sc_rank_body)


_BB = 2


def _build_block(rank_ref, out_ref):
    rank = rank_ref[...]
    r = lax.broadcasted_iota(jnp.int32, (_BB, _N, _N), 1)
    out_ref[...] = jnp.where(rank == r, 1.0, 0.0)


def kernel(scores):
    rank = _sc_rank(scores)
    return pl.pallas_call(
        _build_block,
        grid=(_B // _BB,),
        in_specs=[pl.BlockSpec((_BB, 1, _N), lambda b: (b, 0, 0))],
        out_specs=pl.BlockSpec((_BB, _N, _N), lambda b: (b, 0, 0)),
        out_shape=jax.ShapeDtypeStruct((_B, _N, _N), scores.dtype),
    )(rank[:, None, :])


# speedup vs baseline: 1.9976x; 1.0142x over previous
import functools

import jax
import jax.numpy as jnp
from jax import lax
from jax.experimental import pallas as pl
from jax.experimental.pallas import tpu as pltpu
from jax.experimental.pallas import tpu_sc as plsc

_B, _N = 32, 1024
_L = 16
_V = _N // _L
_D0 = 16


def _sc_rank_body(scores_hbm, rank_hbm, xb, yb, vxb, vyb, rb_):
    wid = lax.axis_index("s") * 2 + lax.axis_index("c")
    pltpu.sync_copy(scores_hbm.at[wid], xb.at[pl.ds(_D0, _N)])

    @plsc.parallel_loop(0, _V)
    def _init(i):
        off = _D0 + i * _L
        k = -xb[pl.ds(off, _L)]
        v = lax.iota(jnp.int32, _L) + i * _L
        sk, sv = plsc.sort_key_val(k, v)
        xb[pl.ds(off, _L)] = sk
        vxb[pl.ds(off, _L)] = sv

    m = _L
    while m < _N:
        k = m // _L

        @plsc.parallel_loop(0, _N // (2 * m))
        def _pair(p, m=m, k=k):
            base = _D0 + p * 2 * m
            for i in range(k):
                ak = xb[pl.ds(base + i * _L, _L)]
                bk = lax.rev(xb[pl.ds(base + m + (k - 1 - i) * _L, _L)], (0,))
                av = vxb[pl.ds(base + i * _L, _L)]
                bv = lax.rev(vxb[pl.ds(base + m + (k - 1 - i) * _L, _L)], (0,))
                sel = ak <= bk
                yb[pl.ds(base + i * _L, _L)] = jnp.minimum(ak, bk)
                yb[pl.ds(base + m + i * _L, _L)] = jnp.maximum(ak, bk)
                vyb[pl.ds(base + i * _L, _L)] = jnp.where(sel, av, bv)
                vyb[pl.ds(base + m + i * _L, _L)] = jnp.where(sel, bv, av)
            dd = k // 2
            while dd >= 1:
                for g in range(k // dd):
                    for i in range(dd):
                        u = base + (g * 2 * dd + i) * _L
                        w = u + dd * _L
                        ak = yb[pl.ds(u, _L)]
                        bk = yb[pl.ds(w, _L)]
                        av = vyb[pl.ds(u, _L)]
                        bv = vyb[pl.ds(w, _L)]
                        sel = ak <= bk
                        yb[pl.ds(u, _L)] = jnp.minimum(ak, bk)
                        yb[pl.ds(w, _L)] = jnp.maximum(ak, bk)
                        vyb[pl.ds(u, _L)] = jnp.where(sel, av, bv)
                        vyb[pl.ds(w, _L)] = jnp.where(sel, bv, av)
                dd //= 2
            for i in range(2 * k):
                kk = yb[pl.ds(base + i * _L, _L)]
                vv = vyb[pl.ds(base + i * _L, _L)]
                sk, sv = plsc.sort_key_val(kk, vv)
                xb[pl.ds(base + i * _L, _L)] = sk
                vxb[pl.ds(base + i * _L, _L)] = sv
        m *= 2

    @plsc.parallel_loop(0, _V)
    def _fin(i):
        pv = vxb[pl.ds(_D0 + i * _L, _L)]
        rv = lax.iota(jnp.int32, _L) + i * _L
        plsc.store_scatter(rb_, [pv], rv)
    pltpu.sync_copy(rb_, rank_hbm.at[wid])


_sc_rank = functools.partial(
    pl.kernel,
    out_type=jax.ShapeDtypeStruct((_B, _N), jnp.int32),
    mesh=plsc.VectorSubcoreMesh(core_axis_name="c", subcore_axis_name="s"),
    compiler_params=pltpu.CompilerParams(
        needs_layout_passes=False, use_tc_tiling_on_sc=False),
    scratch_types=[
        pltpu.VMEM((_D0 + _N,), jnp.float32),
        pltpu.VMEM((_D0 + _N,), jnp.float32),
        pltpu.VMEM((_D0 + _N,), jnp.int32),
        pltpu.VMEM((_D0 + _N,), jnp.int32),
        pltpu.VMEM((_N,), jnp.int32),
    ],
)(_sc_rank_body)


_BB = 1


def _build_block(rank_ref, out_ref):
    rank = rank_ref[...]
    r = lax.broadcasted_iota(jnp.int32, (_BB, _N, _N), 1)
    out_ref[...] = jnp.where(rank == r, 1.0, 0.0)


def kernel(scores):
    rank = _sc_rank(scores)
    return pl.pallas_call(
        _build_block,
        grid=(_B // _BB,),
        in_specs=[pl.BlockSpec((_BB, 1, _N), lambda b: (b, 0, 0))],
        out_specs=pl.BlockSpec((_BB, _N, _N), lambda b: (b, 0, 0)),
        out_shape=jax.ShapeDtypeStruct((_B, _N, _N), scores.dtype),
    )(rank[:, None, :])
